# baseline (speedup 1.0000x reference)
import functools

import jax
import jax.numpy as jnp
from jax import lax
from jax.experimental import pallas as pl
from jax.experimental.pallas import tpu as pltpu
from jax.experimental.pallas import tpu_sc as plsc

H = 768
LANES = 16
NJ = H // LANES
CHUNK = 16
EPS = 1e-8


def _emb_kernel(ids_hbm, wword_hbm, wpos_hbm, wtype_hbm, gamma_hbm, beta_hbm,
                out_hbm, ids_v, in_v, out_v, pos_v, type_v, gamma_v, beta_v,
                g0, p0, o0, g1, p1, o1, *, tokens_per_worker, seq_len):
    nc = 2
    wid = lax.axis_index("s") * nc + lax.axis_index("c")
    base = wid * tokens_per_worker
    nchunks = tokens_per_worker // CHUNK
    sems = ((g0, p0, o0), (g1, p1, o1))

    pltpu.sync_copy(ids_hbm.at[pl.ds(base, tokens_per_worker)], ids_v)
    pltpu.sync_copy(wtype_hbm.at[0], type_v)
    pltpu.sync_copy(gamma_hbm, gamma_v)
    pltpu.sync_copy(beta_hbm, beta_v)

    inv_h = jnp.float32(1.0 / H)
    lane = lax.iota(jnp.int32, LANES)
    bfly = [lane ^ k for k in (8, 4, 2, 1)]

    def allsum(v):
        for idx in bfly:
            v = v + v.at[idx].get(mode="promise_in_bounds")
        return v

    def issue_in(c, b):
        t0 = base + c * CHUNK
        l0 = lax.rem(t0, seq_len)
        pltpu.async_copy(wword_hbm.at[ids_v.at[pl.ds(c * CHUNK, CHUNK)]],
                         in_v.at[b], sems[b][0])
        pltpu.async_copy(wpos_hbm.at[pl.ds(l0, CHUNK)], pos_v.at[b],
                         sems[b][1])

    def wait_in(b):
        pltpu.make_async_copy(wword_hbm.at[pl.ds(0, CHUNK)], in_v.at[b],
                              sems[b][0]).wait()
        pltpu.make_async_copy(wpos_hbm.at[pl.ds(0, CHUNK)], pos_v.at[b],
                              sems[b][1]).wait()

    def issue_out(c, b):
        pltpu.async_copy(out_v.at[b], out_hbm.at[pl.ds(base + c * CHUNK,
                                                       CHUNK)], sems[b][2])

    def wait_out(b):
        pltpu.make_async_copy(out_v.at[b], out_hbm.at[pl.ds(0, CHUNK)],
                              sems[b][2]).wait()

    def compute(b):
        in2d = in_v.at[b]
        pos2d = pos_v.at[b]
        out2d = out_v.at[b]

        def p1_body(i, carry):
            hv, a0, a1, a2, a3, q0, q1, q2, q3 = carry
            accs = [a0, a1, a2, a3]
            sqs = [q0, q1, q2, q3]
            tyb = type_v[pl.ds(i * LANES, LANES)]
            for u in range(LANES):
                ty = tyb.at[jnp.full((LANES,), u, jnp.int32)].get(
                    mode="promise_in_bounds")
                w = plsc.load_gather(in2d, [lane, hv])
                p = plsc.load_gather(pos2d, [lane, hv])
                x = w + p + ty
                plsc.store_scatter(in2d, [lane, hv], x)
                k = u % 4
                accs[k] = accs[k] + x
                sqs[k] = sqs[k] + x * x
                hv = hv + 1
            return (hv, accs[0], accs[1], accs[2], accs[3],
                    sqs[0], sqs[1], sqs[2], sqs[3])

        z = jnp.zeros((LANES,), jnp.float32)
        hv0 = jnp.zeros((LANES,), jnp.int32)
        res = lax.fori_loop(0, NJ, p1_body, (hv0, z, z, z, z, z, z, z, z))
        acc = (res[1] + res[2]) + (res[3] + res[4])
        acc2 = (res[5] + res[6]) + (res[7] + res[8])
        mean = acc * inv_h
        d = acc2 * inv_h - mean * mean + EPS
        iv = plsc.bitcast(d, jnp.int32)
        y = plsc.bitcast(jnp.int32(0x5F3759DF) - (iv >> 1), jnp.float32)
        for _ in range(3):
            y = y * (1.5 - 0.5 * d * y * y)

        def p2_body(i, hv):
            gb = gamma_v[pl.ds(i * LANES, LANES)]
            bb = beta_v[pl.ds(i * LANES, LANES)]
            for u in range(LANES):
                uidx = jnp.full((LANES,), u, jnp.int32)
                gh = gb.at[uidx].get(mode="promise_in_bounds")
                bh = bb.at[uidx].get(mode="promise_in_bounds")
                x = plsc.load_gather(in2d, [lane, hv])
                o = (x - mean) * y * gh + bh
                plsc.store_scatter(out2d, [lane, hv], o)
                hv = hv + 1
            return hv

        lax.fori_loop(0, NJ, p2_body, hv0)

    issue_in(0, 0)
    issue_in(1, 1)
    for b in (0, 1):
        wait_in(b)
        compute(b)
        issue_out(b, b)
        issue_in(b + 2, b)

    def pair_body(i, _):
        for b in (0, 1):
            c = 2 * i + b
            wait_in(b)
            wait_out(b)
            compute(b)
            issue_out(c, b)
            issue_in(c + 2, b)
        return 0

    lax.fori_loop(1, nchunks // 2 - 1, pair_body, 0)

    for b in (0, 1):
        c = nchunks - 2 + b
        wait_in(b)
        wait_out(b)
        compute(b)
        issue_out(c, b)
    for b in (0, 1):
        wait_out(b)


def kernel(input_ids, W_word, W_pos, W_type, gamma, beta):
    B, L = input_ids.shape
    V, Hdim = W_word.shape
    assert Hdim == H
    ids = input_ids.reshape(-1).astype(jnp.int32)
    n_tok = B * L
    nw = 32
    tokens_per_worker = n_tok // nw

    mesh = plsc.VectorSubcoreMesh(core_axis_name="c", subcore_axis_name="s")
    body = functools.partial(_emb_kernel, tokens_per_worker=tokens_per_worker,
                             seq_len=L)
    out = pl.kernel(
        body,
        out_type=jax.ShapeDtypeStruct((n_tok, H), jnp.float32),
        mesh=mesh,
        scratch_types=[
            pltpu.VMEM((tokens_per_worker,), jnp.int32),
            pltpu.VMEM((2, CHUNK, H), jnp.float32),
            pltpu.VMEM((2, CHUNK, H), jnp.float32),
            pltpu.VMEM((2, CHUNK, H), jnp.float32),
            pltpu.VMEM((H,), jnp.float32),
            pltpu.VMEM((H,), jnp.float32),
            pltpu.VMEM((H,), jnp.float32),
            pltpu.SemaphoreType.DMA,
            pltpu.SemaphoreType.DMA,
            pltpu.SemaphoreType.DMA,
            pltpu.SemaphoreType.DMA,
            pltpu.SemaphoreType.DMA,
            pltpu.SemaphoreType.DMA,
        ],
        compiler_params=pltpu.CompilerParams(needs_layout_passes=False),
    )(ids, W_word, W_pos, W_type, gamma, beta)
    return out.reshape(B, L, H)


# speedup vs baseline: 13.5627x; 13.5627x over previous
import functools

import jax
import jax.numpy as jnp
from jax import lax
from jax.experimental import pallas as pl
from jax.experimental.pallas import tpu as pltpu
from jax.experimental.pallas import tpu_sc as plsc

H = 768
LANES = 16
NJ = H // LANES
CHUNK = 16
EPS = 1e-8


def _emb_kernel(ids_hbm, wword_hbm, wpos_hbm, wtype_hbm, gamma_hbm, beta_hbm,
                out_hbm, ids_v, in_v, out_v, pos_v, type_v, gamma_v, beta_v,
                g0, p0, o0, g1, p1, o1, *, tokens_per_worker, seq_len):
    nc = 2
    wid = lax.axis_index("s") * nc + lax.axis_index("c")
    base = wid * tokens_per_worker
    nchunks = tokens_per_worker // CHUNK
    sems = ((g0, p0, o0), (g1, p1, o1))

    pltpu.sync_copy(ids_hbm.at[pl.ds(base, tokens_per_worker)], ids_v)
    pltpu.sync_copy(wtype_hbm.at[0], type_v)
    pltpu.sync_copy(gamma_hbm, gamma_v)
    pltpu.sync_copy(beta_hbm, beta_v)

    inv_h = jnp.float32(1.0 / H)
    lane = lax.iota(jnp.int32, LANES)
    bfly = [lane ^ k for k in (8, 4, 2, 1)]

    def allsum(v):
        for idx in bfly:
            v = v + v.at[idx].get(mode="promise_in_bounds")
        return v

    def issue_in(c, b):
        t0 = base + c * CHUNK
        l0 = lax.rem(t0, seq_len)
        pltpu.async_copy(wword_hbm.at[ids_v.at[pl.ds(c * CHUNK, CHUNK)]],
                         in_v.at[b], sems[b][0])
        pltpu.async_copy(wpos_hbm.at[pl.ds(l0, CHUNK)], pos_v.at[b],
                         sems[b][1])

    def wait_in(b):
        pltpu.make_async_copy(wword_hbm.at[pl.ds(0, CHUNK)], in_v.at[b],
                              sems[b][0]).wait()
        pltpu.make_async_copy(wpos_hbm.at[pl.ds(0, CHUNK)], pos_v.at[b],
                              sems[b][1]).wait()

    def issue_out(c, b):
        pltpu.async_copy(out_v.at[b], out_hbm.at[pl.ds(base + c * CHUNK,
                                                       CHUNK)], sems[b][2])

    def wait_out(b):
        pltpu.make_async_copy(out_v.at[b], out_hbm.at[pl.ds(0, CHUNK)],
                              sems[b][2]).wait()

    def compute(b):
        z = jnp.zeros((LANES,), jnp.float32)

        def p1_body(j, carry):
            jds = pl.ds(j * LANES, LANES)
            accs = list(carry[:CHUNK])
            sqs = list(carry[CHUNK:])
            ty = type_v[jds]
            for t in range(CHUNK):
                x = in_v[b, t, jds] + pos_v[b, t, jds] + ty
                in_v[b, t, jds] = x
                accs[t] = accs[t] + x
                sqs[t] = sqs[t] + x * x
            return tuple(accs) + tuple(sqs)

        res = lax.fori_loop(0, NJ, p1_body, (z,) * (2 * CHUNK))

        means = []
        ys = []
        for t in range(CHUNK):
            s = allsum(res[t])
            q = allsum(res[CHUNK + t])
            mean = s * inv_h
            d = q * inv_h - mean * mean + EPS
            iv = plsc.bitcast(d, jnp.int32)
            y = plsc.bitcast(jnp.int32(0x5F3759DF) - (iv >> 1), jnp.float32)
            for _ in range(3):
                y = y * (1.5 - 0.5 * d * y * y)
            means.append(mean)
            ys.append(y)

        def p2_body(j, carry):
            jds = pl.ds(j * LANES, LANES)
            g = gamma_v[jds]
            bet = beta_v[jds]
            for t in range(CHUNK):
                xm = (in_v[b, t, jds] - carry[t]) * carry[CHUNK + t]
                out_v[b, t, jds] = xm * g + bet
            return carry

        lax.fori_loop(0, NJ, p2_body, tuple(means) + tuple(ys))

    issue_in(0, 0)
    issue_in(1, 1)
    for b in (0, 1):
        wait_in(b)
        compute(b)
        issue_out(b, b)
        issue_in(b + 2, b)

    def pair_body(i, _):
        for b in (0, 1):
            c = 2 * i + b
            wait_in(b)
            wait_out(b)
            compute(b)
            issue_out(c, b)
            issue_in(c + 2, b)
        return 0

    lax.fori_loop(1, nchunks // 2 - 1, pair_body, 0)

    for b in (0, 1):
        c = nchunks - 2 + b
        wait_in(b)
        wait_out(b)
        compute(b)
        issue_out(c, b)
    for b in (0, 1):
        wait_out(b)


def kernel(input_ids, W_word, W_pos, W_type, gamma, beta):
    B, L = input_ids.shape
    V, Hdim = W_word.shape
    assert Hdim == H
    ids = input_ids.reshape(-1).astype(jnp.int32)
    n_tok = B * L
    nw = 32
    tokens_per_worker = n_tok // nw

    mesh = plsc.VectorSubcoreMesh(core_axis_name="c", subcore_axis_name="s")
    body = functools.partial(_emb_kernel, tokens_per_worker=tokens_per_worker,
                             seq_len=L)
    out = pl.kernel(
        body,
        out_type=jax.ShapeDtypeStruct((n_tok, H), jnp.float32),
        mesh=mesh,
        scratch_types=[
            pltpu.VMEM((tokens_per_worker,), jnp.int32),
            pltpu.VMEM((2, CHUNK, H), jnp.float32),
            pltpu.VMEM((2, CHUNK, H), jnp.float32),
            pltpu.VMEM((2, CHUNK, H), jnp.float32),
            pltpu.VMEM((H,), jnp.float32),
            pltpu.VMEM((H,), jnp.float32),
            pltpu.VMEM((H,), jnp.float32),
            pltpu.SemaphoreType.DMA,
            pltpu.SemaphoreType.DMA,
            pltpu.SemaphoreType.DMA,
            pltpu.SemaphoreType.DMA,
            pltpu.SemaphoreType.DMA,
            pltpu.SemaphoreType.DMA,
        ],
        compiler_params=pltpu.CompilerParams(needs_layout_passes=False),
    )(ids, W_word, W_pos, W_type, gamma, beta)
    return out.reshape(B, L, H)
